# baseline (speedup 1.0000x reference)
import math

import jax
import jax.numpy as jnp
import numpy as np
from jax.experimental import pallas as pl
from jax.experimental.pallas import tpu as pltpu
from jax._src.pallas.mosaic import sc_core as plsc_core

N = 2048
HEADS = 16
NUM_BUCKETS = 32
MAX_DISTANCE = 128
WW = 4352
VW = 4096
DW = 4608
NT = N // 128
HALF = DW // 2


def _bucket_indices_np():
    d = np.clip(np.arange(DW, dtype=np.int64) - (129 + N - 1), -(N - 1), N - 1)
    nb = NUM_BUCKETS // 2
    neg = -d
    ret = np.where(neg < 0, nb, 0).astype(np.int32)
    an = np.abs(neg)
    max_exact = nb // 2
    nf = np.maximum(an, 1).astype(np.float32)
    val = max_exact + (
        np.log(nf / np.float32(max_exact)) / np.float32(math.log(MAX_DISTANCE / max_exact)) * (nb - max_exact)
    ).astype(np.int32)
    val = np.minimum(val, nb - 1)
    return (ret + np.where(an < max_exact, an, val)).astype(np.int32)


def _gather16(src, idx):
    dnums = jax.lax.GatherDimensionNumbers(
        offset_dims=(), collapsed_slice_dims=(0,), start_index_map=(0,)
    )
    return jax.lax.gather(
        src, idx[:, None], dnums, (1,),
        mode=jax.lax.GatherScatterMode.PROMISE_IN_BOUNDS,
    )


def _sc_lookup_body(tab_ref, bidx_ref, o_ref, tabv, bidxv, dv, sem):
    c = jax.lax.axis_index("c")
    s = jax.lax.axis_index("s")
    tid = c * 16 + s
    h = jax.lax.div(tid, 2)
    half = jax.lax.rem(tid, 2)
    off = half * HALF

    cp_tab = pltpu.make_async_copy(tab_ref.at[h, :], tabv, sem.at[0])
    cp_tab.start()
    cp_idx = pltpu.make_async_copy(bidx_ref.at[pl.ds(off, HALF)], bidxv, sem.at[1])
    cp_idx.start()
    cp_tab.wait()
    cp_idx.wait()

    tab_lo = tabv[pl.ds(0, 16)]
    tab_hi = tabv[pl.ds(16, 16)]

    def chunk(i, carry):
        idx = bidxv[pl.ds(i * 16, 16)]
        lo = _gather16(tab_lo, jnp.minimum(idx, 15))
        hi = _gather16(tab_hi, jnp.maximum(idx - 16, 0))
        dv[pl.ds(i * 16, 16)] = jnp.where(idx < 16, lo, hi)
        return carry

    jax.lax.fori_loop(0, HALF // 16, chunk, 0)

    cp_out = pltpu.make_async_copy(dv, o_ref.at[h, 0, pl.ds(off, HALF)], sem.at[2])
    cp_out.start()
    cp_out.wait()


def _sc_lookup(table):
    bidx = jnp.asarray(_bucket_indices_np())
    body = pl.kernel(
        _sc_lookup_body,
        out_type=jax.ShapeDtypeStruct((HEADS, 1, DW), jnp.float32),
        mesh=plsc_core.VectorSubcoreMesh(core_axis_name="c", subcore_axis_name="s"),
        scratch_types=[
            pltpu.VMEM((128,), jnp.float32),
            pltpu.VMEM((HALF,), jnp.int32),
            pltpu.VMEM((HALF,), jnp.float32),
            pltpu.SemaphoreType.DMA((3,)),
        ],
    )
    tab_padded = jnp.zeros((HEADS, 128), table.dtype).at[:, :NUM_BUCKETS].set(table.T)
    return body(tab_padded, bidx)


def _build(d_ref, w_ref, v3_ref, r):
    for s in range(8):
        w_ref[s : s + 1, :] = d_ref[0, 0:1, 8 - s : 8 - s + WW]
    for k in range(16):
        v3_ref[r, 8 * k : 8 * k + 8, :] = w_ref[:, 120 - 8 * k : 120 - 8 * k + VW]


def _block_copy(o_ref, v3_ref, sem_ref, h, r, t):
    src = v3_ref.at[r, :, pl.ds((NT - t) * 128, N)]
    dst = o_ref.at[h, pl.ds(128 * t, 128), :]
    return pltpu.make_async_copy(src, dst, sem_ref.at[r])


def _tc_body(dcur_ref, dnext_ref, o_ref, w_ref, v3_ref, sem_ref):
    h = pl.program_id(0)
    r = jax.lax.rem(h, 3)
    rn = jax.lax.rem(h + 1, 3)

    @pl.when(h == 0)
    def _prologue():
        _build(dcur_ref, w_ref, v3_ref, 0)

    for t in range(NT):
        _block_copy(o_ref, v3_ref, sem_ref, h, r, t).start()

    @pl.when(h >= 2)
    def _reclaim():
        for t in range(NT):
            _block_copy(o_ref, v3_ref, sem_ref, h - 2, rn, t).wait()

    @pl.when(h < HEADS - 1)
    def _build_next():
        _build(dnext_ref, w_ref, v3_ref, rn)

    @pl.when(h == HEADS - 1)
    def _drain():
        for t in range(NT):
            _block_copy(o_ref, v3_ref, sem_ref, h - 1, jax.lax.rem(h - 1, 3), t).wait()
        for t in range(NT):
            _block_copy(o_ref, v3_ref, sem_ref, h, r, t).wait()


def kernel(n, relative_attention_bias):
    del n
    d_all = _sc_lookup(relative_attention_bias)
    out = pl.pallas_call(
        _tc_body,
        grid=(HEADS,),
        in_specs=[
            pl.BlockSpec((1, 1, DW), lambda h: (h, 0, 0)),
            pl.BlockSpec((1, 1, DW), lambda h: (jnp.minimum(h + 1, HEADS - 1), 0, 0)),
        ],
        out_specs=pl.BlockSpec(memory_space=pl.ANY),
        out_shape=jax.ShapeDtypeStruct((HEADS, N, N), jnp.float32),
        scratch_shapes=[
            pltpu.VMEM((8, WW), jnp.float32),
            pltpu.VMEM((3, 128, VW), jnp.float32),
            pltpu.SemaphoreType.DMA((3,)),
        ],
    )(d_all, d_all)
    return out


# speedup vs baseline: 1.2716x; 1.2716x over previous
import math

import jax
import jax.numpy as jnp
from jax.experimental import pallas as pl
from jax.experimental.pallas import tpu as pltpu

N = 2048
HEADS = 16
NUM_BUCKETS = 32
MAX_DISTANCE = 128
WW = 4352
VW = 4096
NT = N // 128


def _build(tab_ref, w_ref, v3_ref, hh, r):
    s = jax.lax.broadcasted_iota(jnp.int32, (8, WW), 0)
    z = jax.lax.broadcasted_iota(jnp.int32, (8, WW), 1)
    d = jnp.clip(z - s - (121 + N - 1), -(N - 1), N - 1)
    nb = NUM_BUCKETS // 2
    neg = -d
    ret = jnp.where(neg < 0, nb, 0)
    an = jnp.abs(neg)
    max_exact = nb // 2
    nf = jnp.maximum(an.astype(jnp.float32), 1.0)
    val_large = max_exact + (
        jnp.log(nf / max_exact) / math.log(MAX_DISTANCE / max_exact) * (nb - max_exact)
    ).astype(jnp.int32)
    val_large = jnp.minimum(val_large, nb - 1)
    bucket = ret + jnp.where(an < max_exact, an, val_large)
    acc = jnp.zeros((8, WW), jnp.float32)
    for b in range(NUM_BUCKETS):
        acc = jnp.where(bucket == b, tab_ref[hh, b], acc)
    w_ref[:, :] = acc
    for k in range(16):
        v3_ref[r, 8 * k : 8 * k + 8, :] = w_ref[:, 120 - 8 * k : 120 - 8 * k + VW]


def _block_copy(o_ref, v3_ref, sem_ref, h, r, t):
    src = v3_ref.at[r, :, pl.ds((NT - t) * 128, N)]
    dst = o_ref.at[h, pl.ds(128 * t, 128), :]
    return pltpu.make_async_copy(src, dst, sem_ref.at[r])


def _body(tab_ref, o_ref, w_ref, v3_ref, sem_ref):
    h = pl.program_id(0)
    r = jax.lax.rem(h, 3)
    rn = jax.lax.rem(h + 1, 3)

    @pl.when(h == 0)
    def _prologue():
        _build(tab_ref, w_ref, v3_ref, 0, 0)

    for t in range(NT):
        _block_copy(o_ref, v3_ref, sem_ref, h, r, t).start()

    @pl.when(h >= 2)
    def _reclaim():
        for t in range(NT):
            _block_copy(o_ref, v3_ref, sem_ref, h - 2, rn, t).wait()

    @pl.when(h < HEADS - 1)
    def _build_next():
        _build(tab_ref, w_ref, v3_ref, h + 1, rn)

    @pl.when(h == HEADS - 1)
    def _drain():
        for t in range(NT):
            _block_copy(o_ref, v3_ref, sem_ref, h - 1, jax.lax.rem(h - 1, 3), t).wait()
        for t in range(NT):
            _block_copy(o_ref, v3_ref, sem_ref, h, r, t).wait()


def kernel(n, relative_attention_bias):
    del n
    tab_t = relative_attention_bias.T
    out = pl.pallas_call(
        _body,
        grid=(HEADS,),
        in_specs=[pl.BlockSpec(memory_space=pltpu.SMEM)],
        out_specs=pl.BlockSpec(memory_space=pl.ANY),
        out_shape=jax.ShapeDtypeStruct((HEADS, N, N), jnp.float32),
        scratch_shapes=[
            pltpu.VMEM((8, WW), jnp.float32),
            pltpu.VMEM((3, 128, VW), jnp.float32),
            pltpu.SemaphoreType.DMA((3,)),
        ],
    )(tab_t)
    return out
